# baseline (speedup 1.0000x reference)
import functools

import jax
import jax.numpy as jnp
from jax import lax
from jax.experimental import pallas as pl
from jax.experimental.pallas import tpu as pltpu
from jax.experimental.pallas import tpu_sc as plsc

B = 4096
T = 20
V = 1000
H = 1024
N = B * T

NC = 2
NS = 16
NW = NC * NS
ROWS_W = N // NW
SENT_W = ROWS_W // T
BLK = 16
K = 4
G = ROWS_W // BLK


def _sc_body(x_hbm, z_v_hbm, z_h_hbm,
             sent_hbm, logits_hbm, hidden_hbm,
             tok_v, sent_v, lb0, lb1, lb2, lb3, zbuf, tsave,
             ls0, ls1, ls2, ls3, hs0, hs1, hs2, hs3):
    lbuf = (lb0, lb1, lb2, lb3)
    lsem = (ls0, ls1, ls2, ls3)
    hsem = (hs0, hs1, hs2, hs3)

    wid = lax.axis_index("s") * NC + lax.axis_index("c")
    base = wid * ROWS_W
    lane = lax.iota(jnp.int32, 16)
    onesf = jnp.full((16,), 1.0, jnp.float32)
    zerosf = jnp.zeros((16,), jnp.float32)

    pltpu.sync_copy(x_hbm.at[pl.ds(base, ROWS_W)], tok_v)

    for b in range(K):
        pltpu.sync_copy(z_v_hbm, lbuf[b])
        tsave[b] = jnp.zeros((16,), jnp.int32)
    pltpu.sync_copy(z_h_hbm, zbuf)

    for j in range(SENT_W // 16):
        jbase = j * 16 * T

        def tstep(t, seen, jbase=jbase):
            idx = jbase + lane * T + t
            toki = plsc.load_gather(tok_v, [idx]).astype(jnp.int32)
            seen = seen | (toki == 0).astype(jnp.int32)
            out = jnp.where(seen == 1, 0, toki)
            plsc.store_scatter(sent_v, [idx], out)
            return seen

        lax.fori_loop(0, T, tstep, jnp.zeros((16,), jnp.int32))
    pltpu.sync_copy(sent_v, sent_hbm.at[pl.ds(base, ROWS_W)])

    def block(i, b, wait):
        g = i * K + b
        row0 = base + g * BLK
        if wait:
            pltpu.make_async_copy(z_v_hbm, lbuf[b], lsem[b]).wait()
            pltpu.make_async_copy(z_h_hbm, zbuf, hsem[b]).wait()
        plsc.store_scatter(lbuf[b], [lane, tsave[b]], zerosf)
        toki = plsc.load_gather(tok_v, [g * BLK + lane]).astype(jnp.int32)
        plsc.store_scatter(lbuf[b], [lane, toki], onesf)
        tsave[b] = toki
        pltpu.async_copy(lbuf[b], logits_hbm.at[pl.ds(row0, BLK)], lsem[b])
        pltpu.async_copy(zbuf, hidden_hbm.at[pl.ds(row0, BLK)], hsem[b])

    for b in range(K):
        block(0, b, wait=False)

    def ring(i, _):
        for b in range(K):
            block(i, b, wait=True)
        return 0

    lax.fori_loop(1, G // K, ring, 0)

    for b in range(K):
        pltpu.make_async_copy(z_v_hbm, lbuf[b], lsem[b]).wait()
        pltpu.make_async_copy(z_h_hbm, zbuf, hsem[b]).wait()


@jax.jit
def _run(x_flat, z_v, z_h):
    mesh = plsc.VectorSubcoreMesh(core_axis_name="c", subcore_axis_name="s")
    return pl.kernel(
        _sc_body,
        out_type=[
            jax.ShapeDtypeStruct((N,), jnp.int32),
            jax.ShapeDtypeStruct((N, V), jnp.float32),
            jax.ShapeDtypeStruct((N, H), jnp.float32),
        ],
        mesh=mesh,
        scratch_types=[
            pltpu.VMEM((ROWS_W,), jnp.float32),
            pltpu.VMEM((ROWS_W,), jnp.int32),
            pltpu.VMEM((BLK, V), jnp.float32),
            pltpu.VMEM((BLK, V), jnp.float32),
            pltpu.VMEM((BLK, V), jnp.float32),
            pltpu.VMEM((BLK, V), jnp.float32),
            pltpu.VMEM((BLK, H), jnp.float32),
            pltpu.VMEM((K, 16), jnp.int32),
            pltpu.SemaphoreType.DMA,
            pltpu.SemaphoreType.DMA,
            pltpu.SemaphoreType.DMA,
            pltpu.SemaphoreType.DMA,
            pltpu.SemaphoreType.DMA,
            pltpu.SemaphoreType.DMA,
            pltpu.SemaphoreType.DMA,
            pltpu.SemaphoreType.DMA,
        ],
        compiler_params=pltpu.CompilerParams(needs_layout_passes=False),
        name="onehot_scatter_sc",
    )(x_flat, z_v, z_h)


def kernel(x):
    x_flat = x.reshape(N)
    z_v = jnp.zeros((BLK, V), jnp.float32)
    z_h = jnp.zeros((BLK, H), jnp.float32)
    sent, logits, hidden = _run(x_flat, z_v, z_h)
    return (sent.reshape(B, T),
            logits.reshape(B, T, V),
            hidden.reshape(B, T, H))


# speedup vs baseline: 1.0066x; 1.0066x over previous
import jax
import jax.numpy as jnp
from jax import lax
from jax.experimental import pallas as pl
from jax.experimental.pallas import tpu as pltpu
from jax.experimental.pallas import tpu_sc as plsc

B = 4096
T = 20
V = 1000
H = 1024

NC = 2
NS = 16
NW = NC * NS
SENT_W = B // NW
K = 4
ZS = 8
HB = SENT_W // ZS


def _sc_body(x_hbm, z_v_hbm, z_h_hbm,
             sent_hbm, logits_hbm, hidden_hbm,
             tok_v, sent_v, lb0, lb1, lb2, lb3, tsave, zbuf,
             ls0, ls1, ls2, ls3, hsem):
    lbuf = (lb0, lb1, lb2, lb3)
    lsem = (ls0, ls1, ls2, ls3)

    cid = lax.axis_index("c")
    sid = lax.axis_index("s")
    wid = sid * NC + cid
    s0 = wid * SENT_W
    lane = lax.iota(jnp.int32, 16)
    onesf = jnp.full((16,), 1.0, jnp.float32)
    zerosf = jnp.zeros((16,), jnp.float32)
    tail = lane < (T - 16)
    t2 = jnp.where(tail, lane + 16, 0)

    pltpu.sync_copy(x_hbm.at[pl.ds(s0, SENT_W)], tok_v)

    for b in range(K):
        pltpu.sync_copy(z_v_hbm, lbuf[b])
        tsave[2 * b] = jnp.zeros((16,), jnp.int32)
        tsave[2 * b + 1] = jnp.zeros((16,), jnp.int32)

    @pl.when(sid == 0)
    def _():
        pltpu.sync_copy(z_h_hbm, zbuf)
    plsc.subcore_barrier()

    for i in range(HB):
        pltpu.async_copy(zbuf, hidden_hbm.at[pl.ds(s0 + i * ZS, ZS)], hsem)

    for j in range(SENT_W // 16):
        srow = j * 16 + lane

        def tstep(t, seen, srow=srow):
            tcol = jnp.full((16,), 0, jnp.int32) + t
            toki = plsc.load_gather(tok_v, [srow, tcol]).astype(jnp.int32)
            seen = seen | (toki == 0).astype(jnp.int32)
            out = jnp.where(seen == 1, 0, toki)
            plsc.store_scatter(sent_v, [srow, tcol], out)
            return seen

        lax.fori_loop(0, T, tstep, jnp.zeros((16,), jnp.int32))
    pltpu.sync_copy(sent_v, sent_hbm.at[pl.ds(s0, SENT_W)])

    def block(g, b, wait):
        if wait:
            pltpu.make_async_copy(z_v_hbm, lbuf[b], lsem[b]).wait()
        plsc.store_scatter(lbuf[b], [lane, tsave[2 * b]], zerosf)
        plsc.store_scatter(lbuf[b], [t2, tsave[2 * b + 1]], zerosf,
                           mask=tail)
        srow = jnp.full((16,), 0, jnp.int32) + g
        tok1 = plsc.load_gather(tok_v, [srow, lane]).astype(jnp.int32)
        tok2 = plsc.load_gather(tok_v, [srow, t2],
                                mask=tail).astype(jnp.int32)
        tok2 = jnp.where(tail, tok2, 0)
        plsc.store_scatter(lbuf[b], [lane, tok1], onesf)
        plsc.store_scatter(lbuf[b], [t2, tok2], onesf, mask=tail)
        tsave[2 * b] = tok1
        tsave[2 * b + 1] = tok2
        pltpu.async_copy(lbuf[b], logits_hbm.at[s0 + g], lsem[b])

    for b in range(K):
        block(b, b, wait=False)

    def ring(i, _):
        for b in range(K):
            block(i * K + b, b, wait=True)
        return 0

    lax.fori_loop(1, SENT_W // K, ring, 0)

    for b in range(K):
        pltpu.make_async_copy(z_v_hbm, lbuf[b], lsem[b]).wait()
    pltpu.make_async_copy(z_h_hbm, zbuf, hsem).wait()
    for _ in range(HB - 1):
        pltpu.make_async_copy(z_h_hbm, zbuf, hsem).wait()


@jax.jit
def _run(x, z_v, z_h):
    mesh = plsc.VectorSubcoreMesh(core_axis_name="c", subcore_axis_name="s")
    return pl.kernel(
        _sc_body,
        out_type=[
            jax.ShapeDtypeStruct((B, T), jnp.int32),
            jax.ShapeDtypeStruct((B, T, V), jnp.float32),
            jax.ShapeDtypeStruct((B, T, H), jnp.float32),
        ],
        mesh=mesh,
        scratch_types=[
            pltpu.VMEM((SENT_W, T), jnp.float32),
            pltpu.VMEM((SENT_W, T), jnp.int32),
            pltpu.VMEM((T, V), jnp.float32),
            pltpu.VMEM((T, V), jnp.float32),
            pltpu.VMEM((T, V), jnp.float32),
            pltpu.VMEM((T, V), jnp.float32),
            pltpu.VMEM((2 * K, 16), jnp.int32),
            pltpu.VMEM_SHARED((ZS, T, H), jnp.float32),
            pltpu.SemaphoreType.DMA,
            pltpu.SemaphoreType.DMA,
            pltpu.SemaphoreType.DMA,
            pltpu.SemaphoreType.DMA,
            pltpu.SemaphoreType.DMA,
        ],
        compiler_params=pltpu.CompilerParams(
            needs_layout_passes=False, use_tc_tiling_on_sc=False),
        name="onehot_scatter_sc",
    )(x, z_v, z_h)


def kernel(x):
    z_v = jnp.zeros((T, V), jnp.float32)
    z_h = jnp.zeros((ZS, T, H), jnp.float32)
    sent, logits, hidden = _run(x, z_v, z_h)
    return sent, logits, hidden


# speedup vs baseline: 1.5192x; 1.5093x over previous
import jax
import jax.numpy as jnp
from jax import lax
from jax.experimental import pallas as pl
from jax.experimental.pallas import tpu as pltpu
from jax.experimental.pallas import tpu_sc as plsc

B = 4096
T = 20
V = 1000
H = 1024

NC = 2
NS = 16
NW = NC * NS
SENT_W = B // NW
K = 2
ZS = 8
HB = SENT_W // ZS


def _sc_body(x_hbm, z_v_hbm, z_h_hbm,
             sent_hbm, logits_hbm, hidden_hbm,
             tok_v, sent_v, lb0, lb1, tsave, zbuf,
             ls0, ls1, hsem):
    lbuf = (lb0, lb1)
    lsem = (ls0, ls1)

    cid = lax.axis_index("c")
    sid = lax.axis_index("s")
    wid = sid * NC + cid
    s0 = wid * SENT_W
    lane = lax.iota(jnp.int32, 16)
    onesf = jnp.full((16,), 1.0, jnp.float32)
    zerosf = jnp.zeros((16,), jnp.float32)
    tail = lane < (T - 16)
    t2 = jnp.where(tail, lane + 16, 0)

    pltpu.sync_copy(x_hbm.at[pl.ds(s0, SENT_W)], tok_v)

    for b in range(K):
        pltpu.sync_copy(z_v_hbm, lbuf[b])
        tsave[2 * b] = jnp.zeros((16,), jnp.int32)
        tsave[2 * b + 1] = jnp.zeros((16,), jnp.int32)

    @pl.when(sid == 0)
    def _():
        pltpu.sync_copy(z_h_hbm, zbuf)
    plsc.subcore_barrier()

    for i in range(HB):
        pltpu.async_copy(zbuf, hidden_hbm.at[pl.ds(s0 + i * ZS, ZS)], hsem)

    for j in range(SENT_W // 16):
        srow = j * 16 + lane

        def tstep(t, seen, srow=srow):
            tcol = jnp.full((16,), 0, jnp.int32) + t
            toki = plsc.load_gather(tok_v, [srow, tcol]).astype(jnp.int32)
            seen = seen | (toki == 0).astype(jnp.int32)
            out = jnp.where(seen == 1, 0, toki)
            plsc.store_scatter(sent_v, [srow, tcol], out)
            return seen

        lax.fori_loop(0, T, tstep, jnp.zeros((16,), jnp.int32))
    pltpu.sync_copy(sent_v, sent_hbm.at[pl.ds(s0, SENT_W)])

    def block(g, b, wait):
        if wait:
            pltpu.make_async_copy(z_v_hbm, lbuf[b], lsem[b]).wait()
        plsc.store_scatter(lbuf[b], [lane, tsave[2 * b]], zerosf)
        plsc.store_scatter(lbuf[b], [t2, tsave[2 * b + 1]], zerosf,
                           mask=tail)
        srow = jnp.full((16,), 0, jnp.int32) + g
        tok1 = plsc.load_gather(tok_v, [srow, lane]).astype(jnp.int32)
        tok2 = plsc.load_gather(tok_v, [srow, t2],
                                mask=tail).astype(jnp.int32)
        tok2 = jnp.where(tail, tok2, 0)
        plsc.store_scatter(lbuf[b], [lane, tok1], onesf)
        plsc.store_scatter(lbuf[b], [t2, tok2], onesf, mask=tail)
        tsave[2 * b] = tok1
        tsave[2 * b + 1] = tok2
        pltpu.async_copy(lbuf[b], logits_hbm.at[s0 + g], lsem[b])

    for b in range(K):
        block(b, b, wait=False)

    def ring(i, _):
        for b in range(K):
            block(i * K + b, b, wait=True)
        return 0

    lax.fori_loop(1, SENT_W // K, ring, 0)

    for b in range(K):
        pltpu.make_async_copy(z_v_hbm, lbuf[b], lsem[b]).wait()
    pltpu.make_async_copy(z_h_hbm, zbuf, hsem).wait()
    for _ in range(HB - 1):
        pltpu.make_async_copy(z_h_hbm, zbuf, hsem).wait()


@jax.jit
def _run(x, z_v, z_h):
    mesh = plsc.VectorSubcoreMesh(core_axis_name="c", subcore_axis_name="s")
    return pl.kernel(
        _sc_body,
        out_type=[
            jax.ShapeDtypeStruct((B, T), jnp.int32),
            jax.ShapeDtypeStruct((B, T, V), jnp.float32),
            jax.ShapeDtypeStruct((B, T, H), jnp.float32),
        ],
        mesh=mesh,
        scratch_types=[
            pltpu.VMEM((SENT_W, T), jnp.float32),
            pltpu.VMEM((SENT_W, T), jnp.int32),
            pltpu.VMEM((T, V), jnp.float32),
            pltpu.VMEM((T, V), jnp.float32),
            pltpu.VMEM((2 * K, 16), jnp.int32),
            pltpu.VMEM_SHARED((ZS, T, H), jnp.float32),
            pltpu.SemaphoreType.DMA,
            pltpu.SemaphoreType.DMA,
            pltpu.SemaphoreType.DMA,
        ],
        compiler_params=pltpu.CompilerParams(
            needs_layout_passes=False, use_tc_tiling_on_sc=True),
        name="onehot_scatter_sc",
    )(x, z_v, z_h)


def kernel(x):
    z_v = jnp.zeros((T, V), jnp.float32)
    z_h = jnp.zeros((ZS, T, H), jnp.float32)
    sent, logits, hidden = _run(x, z_v, z_h)
    return sent, logits, hidden


# speedup vs baseline: 1.5533x; 1.0224x over previous
import jax
import jax.numpy as jnp
from jax import lax
from jax.experimental import pallas as pl
from jax.experimental.pallas import tpu as pltpu
from jax.experimental.pallas import tpu_sc as plsc

B = 4096
T = 20
V = 1000
H = 1024

NC = 2
NS = 16
NW = NC * NS
SENT_W = B // NW
K = 2
HBB = 32


def _sc_body(x_hbm, z_v_hbm,
             sent_hbm, logits_hbm,
             tok_v, sent_v, lb0, lb1, tsave,
             ls0, ls1):
    lbuf = (lb0, lb1)
    lsem = (ls0, ls1)

    cid = lax.axis_index("c")
    sid = lax.axis_index("s")
    wid = sid * NC + cid
    s0 = wid * SENT_W
    lane = lax.iota(jnp.int32, 16)
    onesf = jnp.full((16,), 1.0, jnp.float32)
    zerosf = jnp.zeros((16,), jnp.float32)
    tail = lane < (T - 16)
    t2 = jnp.where(tail, lane + 16, 0)

    pltpu.sync_copy(x_hbm.at[pl.ds(s0, SENT_W)], tok_v)

    for b in range(K):
        pltpu.sync_copy(z_v_hbm, lbuf[b])
        tsave[2 * b] = jnp.zeros((16,), jnp.int32)
        tsave[2 * b + 1] = jnp.zeros((16,), jnp.int32)

    for j in range(SENT_W // 16):
        srow = j * 16 + lane

        def tstep(t, seen, srow=srow):
            tcol = jnp.full((16,), 0, jnp.int32) + t
            toki = plsc.load_gather(tok_v, [srow, tcol]).astype(jnp.int32)
            seen = seen | (toki == 0).astype(jnp.int32)
            out = jnp.where(seen == 1, 0, toki)
            plsc.store_scatter(sent_v, [srow, tcol], out)
            return seen

        lax.fori_loop(0, T, tstep, jnp.zeros((16,), jnp.int32))
    pltpu.sync_copy(sent_v, sent_hbm.at[pl.ds(s0, SENT_W)])

    def block(g, b, wait):
        if wait:
            pltpu.make_async_copy(z_v_hbm, lbuf[b], lsem[b]).wait()
        plsc.store_scatter(lbuf[b], [lane, tsave[2 * b]], zerosf)
        plsc.store_scatter(lbuf[b], [t2, tsave[2 * b + 1]], zerosf,
                           mask=tail)
        srow = jnp.full((16,), 0, jnp.int32) + g
        tok1 = plsc.load_gather(tok_v, [srow, lane]).astype(jnp.int32)
        tok2 = plsc.load_gather(tok_v, [srow, t2],
                                mask=tail).astype(jnp.int32)
        tok2 = jnp.where(tail, tok2, 0)
        plsc.store_scatter(lbuf[b], [lane, tok1], onesf)
        plsc.store_scatter(lbuf[b], [t2, tok2], onesf, mask=tail)
        tsave[2 * b] = tok1
        tsave[2 * b + 1] = tok2
        pltpu.async_copy(lbuf[b], logits_hbm.at[s0 + g], lsem[b])

    for b in range(K):
        block(b, b, wait=False)

    def ring(i, _):
        for b in range(K):
            block(i * K + b, b, wait=True)
        return 0

    lax.fori_loop(1, SENT_W // K, ring, 0)

    for b in range(K):
        pltpu.make_async_copy(z_v_hbm, lbuf[b], lsem[b]).wait()


def _tc_zero_body(o_ref):
    o_ref[...] = jnp.zeros_like(o_ref)


@jax.jit
def _run(x, z_v):
    mesh = plsc.VectorSubcoreMesh(core_axis_name="c", subcore_axis_name="s")
    sent, logits = pl.kernel(
        _sc_body,
        out_type=[
            jax.ShapeDtypeStruct((B, T), jnp.int32),
            jax.ShapeDtypeStruct((B, T, V), jnp.float32),
        ],
        mesh=mesh,
        scratch_types=[
            pltpu.VMEM((SENT_W, T), jnp.float32),
            pltpu.VMEM((SENT_W, T), jnp.int32),
            pltpu.VMEM((T, V), jnp.float32),
            pltpu.VMEM((T, V), jnp.float32),
            pltpu.VMEM((2 * K, 16), jnp.int32),
            pltpu.SemaphoreType.DMA,
            pltpu.SemaphoreType.DMA,
        ],
        compiler_params=pltpu.CompilerParams(
            needs_layout_passes=False, use_tc_tiling_on_sc=True),
        name="onehot_scatter_sc",
    )(x, z_v)

    hidden = pl.pallas_call(
        _tc_zero_body,
        out_shape=jax.ShapeDtypeStruct((B, T, H), jnp.float32),
        grid=(B // HBB,),
        out_specs=pl.BlockSpec((HBB, T, H), lambda i: (i, 0, 0)),
        name="hidden_zeros_tc",
    )()
    return sent, logits, hidden


def kernel(x):
    z_v = jnp.zeros((T, V), jnp.float32)
    sent, logits, hidden = _run(x, z_v)
    return sent, logits, hidden


# speedup vs baseline: 5.7087x; 3.6753x over previous
import jax
import jax.numpy as jnp
from jax import lax
from jax.experimental import pallas as pl
from jax.experimental.pallas import tpu as pltpu
from jax.experimental.pallas import tpu_sc as plsc

B = 4096
T = 20
V = 1000
H = 1024

NC = 2
NS = 16
NW = NC * NS
SENT_W = B // NW
VC = 200


def _sc_body(x_hbm, z_h_hbm, sent_hbm, hidden_hbm,
             tok_v, sent_v, zbuf, hsem):
    cid = lax.axis_index("c")
    sid = lax.axis_index("s")
    wid = sid * NC + cid
    s0 = wid * SENT_W
    lane = lax.iota(jnp.int32, 16)

    pltpu.sync_copy(x_hbm.at[:, pl.ds(s0, SENT_W)], tok_v)

    @pl.when(sid == 0)
    def _():
        pltpu.sync_copy(z_h_hbm, zbuf)
    plsc.subcore_barrier()

    for t in range(T):
        pltpu.async_copy(zbuf, hidden_hbm.at[t, pl.ds(s0, SENT_W)], hsem)

    for j in range(SENT_W // 16):
        srow = j * 16 + lane

        def tstep(t, seen, srow=srow):
            tcol = jnp.full((16,), 0, jnp.int32) + t
            toki = plsc.load_gather(tok_v, [tcol, srow]).astype(jnp.int32)
            seen = seen | (toki == 0).astype(jnp.int32)
            out = jnp.where(seen == 1, 0, toki)
            plsc.store_scatter(sent_v, [tcol, srow], out)
            return seen

        lax.fori_loop(0, T, tstep, jnp.zeros((16,), jnp.int32))
    pltpu.sync_copy(sent_v, sent_hbm.at[:, pl.ds(s0, SENT_W)])

    for t in range(T):
        pltpu.make_async_copy(zbuf, hidden_hbm.at[t, pl.ds(s0, SENT_W)],
                              hsem).wait()


def _tc_onehot_body(tok_ref, o_ref):
    t = pl.program_id(0)
    vc = pl.program_id(1)
    tok = tok_ref[t, :].astype(jnp.int32)
    vio = lax.broadcasted_iota(jnp.int32, (VC, B), 0) + vc * VC
    o_ref[0] = jnp.where(vio == tok[None, :], 1.0, 0.0)


@jax.jit
def _run(x_tb, z_h):
    mesh = plsc.VectorSubcoreMesh(core_axis_name="c", subcore_axis_name="s")
    sent_tb, hidden_tbh = pl.kernel(
        _sc_body,
        out_type=[
            jax.ShapeDtypeStruct((T, B), jnp.int32),
            jax.ShapeDtypeStruct((T, B, H), jnp.float32),
        ],
        mesh=mesh,
        scratch_types=[
            pltpu.VMEM((T, SENT_W), jnp.float32),
            pltpu.VMEM((T, SENT_W), jnp.int32),
            pltpu.VMEM_SHARED((SENT_W, H), jnp.float32),
            pltpu.SemaphoreType.DMA,
        ],
        compiler_params=pltpu.CompilerParams(
            needs_layout_passes=False, use_tc_tiling_on_sc=True),
        name="eos_hidden_sc",
    )(x_tb, z_h)

    logits_tvb = pl.pallas_call(
        _tc_onehot_body,
        out_shape=jax.ShapeDtypeStruct((T, V, B), jnp.float32),
        grid=(T, V // VC),
        in_specs=[pl.BlockSpec((T, B), lambda t, v: (0, 0))],
        out_specs=pl.BlockSpec((1, VC, B), lambda t, v: (t, v, 0)),
        name="onehot_tc",
    )(x_tb)
    return sent_tb, logits_tvb, hidden_tbh


def kernel(x):
    x_tb = jnp.transpose(x, (1, 0))
    z_h = jnp.zeros((SENT_W, H), jnp.float32)
    sent_tb, logits_tvb, hidden_tbh = _run(x_tb, z_h)
    return (jnp.transpose(sent_tb, (1, 0)),
            jnp.transpose(logits_tvb, (2, 0, 1)),
            jnp.transpose(hidden_tbh, (1, 0, 2)))


# speedup vs baseline: 5.7255x; 1.0029x over previous
import jax
import jax.numpy as jnp
from jax import lax
from jax.experimental import pallas as pl
from jax.experimental.pallas import tpu as pltpu
from jax.experimental.pallas import tpu_sc as plsc

B = 4096
T = 20
V = 1000
H = 1024

NC = 2
NS = 16
NW = NC * NS
SENT_W = B // NW
VC = 200


def _sc_body(x_hbm, z_h_hbm, sent_hbm, hidden_hbm,
             tok_v, sent_v, zbuf, hsem):
    cid = lax.axis_index("c")
    sid = lax.axis_index("s")
    wid = sid * NC + cid
    s0 = wid * SENT_W
    lane = lax.iota(jnp.int32, 16)

    @pl.when(sid == 0)
    def _():
        pltpu.sync_copy(z_h_hbm, zbuf)
    plsc.subcore_barrier()

    for t in range(T):
        pltpu.async_copy(zbuf, hidden_hbm.at[t, pl.ds(s0, SENT_W)], hsem)

    pltpu.sync_copy(x_hbm.at[:, pl.ds(s0, SENT_W)], tok_v)

    for j in range(SENT_W // 16):
        srow = j * 16 + lane

        def tstep(t, seen, srow=srow):
            tcol = jnp.full((16,), 0, jnp.int32) + t
            toki = plsc.load_gather(tok_v, [tcol, srow]).astype(jnp.int32)
            seen = seen | (toki == 0).astype(jnp.int32)
            out = jnp.where(seen == 1, 0, toki)
            plsc.store_scatter(sent_v, [tcol, srow], out)
            return seen

        lax.fori_loop(0, T, tstep, jnp.zeros((16,), jnp.int32))
    pltpu.sync_copy(sent_v, sent_hbm.at[:, pl.ds(s0, SENT_W)])

    for t in range(T):
        pltpu.make_async_copy(zbuf, hidden_hbm.at[t, pl.ds(s0, SENT_W)],
                              hsem).wait()


def _tc_onehot_body(tok_ref, o_ref):
    t = pl.program_id(0)
    vc = pl.program_id(1)
    tok = tok_ref[t, :].astype(jnp.int32)
    vio = lax.broadcasted_iota(jnp.int32, (VC, B), 0) + vc * VC
    o_ref[0] = jnp.where(vio == tok[None, :], 1.0, 0.0)


@jax.jit
def _run(x_tb, z_h):
    mesh = plsc.VectorSubcoreMesh(core_axis_name="c", subcore_axis_name="s")
    sent_tb, hidden_tbh = pl.kernel(
        _sc_body,
        out_type=[
            jax.ShapeDtypeStruct((T, B), jnp.int32),
            jax.ShapeDtypeStruct((T, B, H), jnp.float32),
        ],
        mesh=mesh,
        scratch_types=[
            pltpu.VMEM((T, SENT_W), jnp.float32),
            pltpu.VMEM((T, SENT_W), jnp.int32),
            pltpu.VMEM_SHARED((SENT_W, H), jnp.float32),
            pltpu.SemaphoreType.DMA,
        ],
        compiler_params=pltpu.CompilerParams(
            needs_layout_passes=False, use_tc_tiling_on_sc=True),
        name="eos_hidden_sc",
    )(x_tb, z_h)

    logits_tvb = pl.pallas_call(
        _tc_onehot_body,
        out_shape=jax.ShapeDtypeStruct((T, V, B), jnp.float32),
        grid=(T, V // VC),
        in_specs=[pl.BlockSpec((T, B), lambda t, v: (0, 0))],
        out_specs=pl.BlockSpec((1, VC, B), lambda t, v: (t, v, 0)),
        name="onehot_tc",
    )(x_tb)
    return sent_tb, logits_tvb, hidden_tbh


def kernel(x):
    x_tb = jnp.transpose(x, (1, 0))
    z_h = jnp.zeros((SENT_W, H), jnp.float32)
    sent_tb, logits_tvb, hidden_tbh = _run(x_tb, z_h)
    return (jnp.transpose(sent_tb, (1, 0)),
            jnp.transpose(logits_tvb, (2, 0, 1)),
            jnp.transpose(hidden_tbh, (1, 0, 2)))


# speedup vs baseline: 5.8104x; 1.0148x over previous
import jax
import jax.numpy as jnp
from jax import lax
from jax.experimental import pallas as pl
from jax.experimental.pallas import tpu as pltpu
from jax.experimental.pallas import tpu_sc as plsc

B = 4096
T = 20
V = 1000
H = 1024

NC = 2
NS = 16
NW = NC * NS
SENT_W = B // NW
VC = 200
HZ = 64


def _sc_body(x_hbm, z_h_hbm, sent_hbm, hidden_hbm,
             tok_v, sent_v, zbuf, hsem):
    cid = lax.axis_index("c")
    sid = lax.axis_index("s")
    wid = sid * NC + cid
    s0 = wid * SENT_W
    lane = lax.iota(jnp.int32, 16)

    pltpu.sync_copy(z_h_hbm, zbuf)

    for t in range(T):
        for k in range(SENT_W // HZ):
            pltpu.async_copy(zbuf, hidden_hbm.at[t, pl.ds(s0 + k * HZ, HZ)],
                             hsem)

    pltpu.sync_copy(x_hbm.at[:, pl.ds(s0, SENT_W)], tok_v)

    for j in range(SENT_W // 16):
        srow = j * 16 + lane

        def tstep(t, seen, srow=srow):
            tcol = jnp.full((16,), 0, jnp.int32) + t
            toki = plsc.load_gather(tok_v, [tcol, srow]).astype(jnp.int32)
            seen = seen | (toki == 0).astype(jnp.int32)
            out = jnp.where(seen == 1, 0, toki)
            plsc.store_scatter(sent_v, [tcol, srow], out)
            return seen

        lax.fori_loop(0, T, tstep, jnp.zeros((16,), jnp.int32))
    pltpu.sync_copy(sent_v, sent_hbm.at[:, pl.ds(s0, SENT_W)])

    for t in range(T):
        for k in range(SENT_W // HZ):
            pltpu.make_async_copy(zbuf,
                                  hidden_hbm.at[t, pl.ds(s0 + k * HZ, HZ)],
                                  hsem).wait()


def _tc_onehot_body(tok_ref, o_ref):
    t = pl.program_id(0)
    vc = pl.program_id(1)
    tok = tok_ref[t, :].astype(jnp.int32)
    vio = lax.broadcasted_iota(jnp.int32, (VC, B), 0) + vc * VC
    o_ref[0] = jnp.where(vio == tok[None, :], 1.0, 0.0)


@jax.jit
def _run(x_tb, z_h):
    mesh = plsc.VectorSubcoreMesh(core_axis_name="c", subcore_axis_name="s")
    sent_tb, hidden_tbh = pl.kernel(
        _sc_body,
        out_type=[
            jax.ShapeDtypeStruct((T, B), jnp.int32),
            jax.ShapeDtypeStruct((T, B, H), jnp.float32),
        ],
        mesh=mesh,
        scratch_types=[
            pltpu.VMEM((T, SENT_W), jnp.float32),
            pltpu.VMEM((T, SENT_W), jnp.int32),
            pltpu.VMEM((HZ, H), jnp.float32),
            pltpu.SemaphoreType.DMA,
        ],
        compiler_params=pltpu.CompilerParams(
            needs_layout_passes=False, use_tc_tiling_on_sc=True),
        name="eos_hidden_sc",
    )(x_tb, z_h)

    logits_tvb = pl.pallas_call(
        _tc_onehot_body,
        out_shape=jax.ShapeDtypeStruct((T, V, B), jnp.float32),
        grid=(T, V // VC),
        in_specs=[pl.BlockSpec((T, B), lambda t, v: (0, 0))],
        out_specs=pl.BlockSpec((1, VC, B), lambda t, v: (t, v, 0)),
        name="onehot_tc",
    )(x_tb)
    return sent_tb, logits_tvb, hidden_tbh


def kernel(x):
    x_tb = jnp.transpose(x, (1, 0))
    z_h = jnp.zeros((HZ, H), jnp.float32)
    sent_tb, logits_tvb, hidden_tbh = _run(x_tb, z_h)
    return (jnp.transpose(sent_tb, (1, 0)),
            jnp.transpose(logits_tvb, (2, 0, 1)),
            jnp.transpose(hidden_tbh, (1, 0, 2)))
